# baseline (speedup 1.0000x reference)
import jax
import jax.numpy as jnp
from jax.experimental import pallas as pl
from jax.experimental.pallas import tpu as pltpu

B = 64
H = 512
W = 512
SIZE = 511
N = SIZE * SIZE
OUT = 128
C = 4096
G = 64
STRIP = 5120


def _fused_kernel(wc_ref, x_ref, wfc_ref, b_ref, out_ref,
                  xcarry_ref, ycarry_ref, acc_ref):
    i = pl.program_id(0)
    j = (G - 1) - i
    shift = 8 * j

    @pl.when(i == 0)
    def _init():
        xcarry_ref[...] = jnp.zeros_like(xcarry_ref)
        ycarry_ref[...] = jnp.zeros_like(ycarry_ref)
        acc_ref[...] = jnp.zeros_like(acc_ref)

    wcv = wc_ref[...]
    w00 = wcv[0, 0]
    w01 = wcv[0, 1]
    w10 = wcv[0, 2]
    w11 = wcv[0, 3]

    xcarry = xcarry_ref[...]
    ycarry = ycarry_ref[...]

    def conv_row(top, bot):
        return jnp.tanh(w00 * top[:, :SIZE] + w01 * top[:, 1:]
                        + w10 * bot[:, :SIZE] + w11 * bot[:, 1:])

    rows = [x_ref[:, d, :] for d in range(8)] + [xcarry]
    y = [conv_row(rows[d], rows[d + 1]) for d in range(8)]

    strip = jnp.concatenate(
        y + [ycarry, jnp.zeros((B, STRIP - 8 * SIZE - 2 * SIZE),
                               dtype=jnp.float32)], axis=1)
    chunk = pltpu.roll(strip, jax.lax.rem(STRIP - shift, STRIP), 1)[:, :C]

    lane = jax.lax.broadcasted_iota(jnp.int32, (1, C), 1)
    chunk = jnp.where(lane < (N - C * j), chunk, 0.0)

    wblk = wfc_ref[...]

    @pl.when(i == 0)
    def _acc_masked():
        wm = jnp.where(lane < (N - C * j), wblk, 0.0)
        acc_ref[...] += jax.lax.dot_general(
            chunk, wm, (((1,), (1,)), ((), ())),
            preferred_element_type=jnp.float32)

    @pl.when(i != 0)
    def _acc():
        acc_ref[...] += jax.lax.dot_general(
            chunk, wblk, (((1,), (1,)), ((), ())),
            preferred_element_type=jnp.float32)

    ycarry_ref[...] = jnp.concatenate([y[0], y[1]], axis=1)
    xcarry_ref[...] = x_ref[:, 0, :]

    @pl.when(i == G - 1)
    def _finalize():
        out_ref[...] = jnp.tanh(acc_ref[...] + b_ref[...])


def kernel(x, W_conv, W_fc, b_fc):
    wc = W_conv.reshape(1, 4)
    b2 = b_fc.reshape(1, OUT)
    return pl.pallas_call(
        _fused_kernel,
        grid=(G,),
        in_specs=[
            pl.BlockSpec((1, 4), lambda i: (0, 0)),
            pl.BlockSpec((B, 8, W), lambda i: (0, G - 1 - i, 0)),
            pl.BlockSpec((OUT, C), lambda i: (0, G - 1 - i)),
            pl.BlockSpec((1, OUT), lambda i: (0, 0)),
        ],
        out_specs=pl.BlockSpec((B, OUT), lambda i: (0, 0)),
        out_shape=jax.ShapeDtypeStruct((B, OUT), jnp.float32),
        scratch_shapes=[
            pltpu.VMEM((B, W), jnp.float32),
            pltpu.VMEM((B, 2 * SIZE), jnp.float32),
            pltpu.VMEM((B, OUT), jnp.float32),
        ],
    )(wc, x, W_fc, b2)


# speedup vs baseline: 1.0651x; 1.0651x over previous
import jax
import jax.numpy as jnp
from jax.experimental import pallas as pl
from jax.experimental.pallas import tpu as pltpu

B = 64
H = 512
W = 512
SIZE = 511
N = SIZE * SIZE
OUT = 128
C = 4096
G = 64
STRIP = 5120


def _fused_kernel(wc_ref, x_ref, wfc_ref, b_ref, out_ref,
                  xcarry_ref, ycarry_ref, strip_ref, acc_ref):
    i = pl.program_id(0)
    j = (G - 1) - i
    shift = 8 * j

    @pl.when(i == 0)
    def _init():
        xcarry_ref[...] = jnp.zeros_like(xcarry_ref)
        ycarry_ref[...] = jnp.zeros_like(ycarry_ref)
        acc_ref[...] = jnp.zeros_like(acc_ref)

    wcv = wc_ref[...]
    w00 = wcv[0, 0]
    w01 = wcv[0, 1]
    w10 = wcv[0, 2]
    w11 = wcv[0, 3]

    xcarry = xcarry_ref[...]
    ycarry = ycarry_ref[...]

    def conv_row(top, bot):
        return jnp.tanh(w00 * top[:, :SIZE] + w01 * top[:, 1:]
                        + w10 * bot[:, :SIZE] + w11 * bot[:, 1:])

    rows = [x_ref[:, d, :] for d in range(8)] + [xcarry]
    y = [conv_row(rows[d], rows[d + 1]) for d in range(8)]

    for d in range(8):
        strip_ref[:, d * SIZE:(d + 1) * SIZE] = y[d]
    strip_ref[:, 8 * SIZE:10 * SIZE] = ycarry
    strip = strip_ref[...]
    chunk = pltpu.roll(strip, jax.lax.rem(STRIP - shift, STRIP), 1)[:, :C]

    wblk = wfc_ref[...]

    @pl.when(i == 0)
    def _acc_masked():
        lane = jax.lax.broadcasted_iota(jnp.int32, (1, C), 1)
        wm = jnp.where(lane < (N - C * j), wblk, 0.0)
        acc_ref[...] += jax.lax.dot_general(
            chunk, wm, (((1,), (1,)), ((), ())),
            preferred_element_type=jnp.float32)

    @pl.when(i != 0)
    def _acc():
        acc_ref[...] += jax.lax.dot_general(
            chunk, wblk, (((1,), (1,)), ((), ())),
            preferred_element_type=jnp.float32)

    ycarry_ref[...] = jnp.concatenate([y[0], y[1]], axis=1)
    xcarry_ref[...] = x_ref[:, 0, :]

    @pl.when(i == G - 1)
    def _finalize():
        out_ref[...] = jnp.tanh(acc_ref[...] + b_ref[...])


def kernel(x, W_conv, W_fc, b_fc):
    wc = W_conv.reshape(1, 4)
    b2 = b_fc.reshape(1, OUT)
    return pl.pallas_call(
        _fused_kernel,
        grid=(G,),
        in_specs=[
            pl.BlockSpec((1, 4), lambda i: (0, 0)),
            pl.BlockSpec((B, 8, W), lambda i: (0, G - 1 - i, 0)),
            pl.BlockSpec((OUT, C), lambda i: (0, G - 1 - i)),
            pl.BlockSpec((1, OUT), lambda i: (0, 0)),
        ],
        out_specs=pl.BlockSpec((B, OUT), lambda i: (0, 0)),
        out_shape=jax.ShapeDtypeStruct((B, OUT), jnp.float32),
        scratch_shapes=[
            pltpu.VMEM((B, W), jnp.float32),
            pltpu.VMEM((B, 2 * SIZE), jnp.float32),
            pltpu.VMEM((B, STRIP), jnp.float32),
            pltpu.VMEM((B, OUT), jnp.float32),
        ],
    )(wc, x, W_fc, b2)


# speedup vs baseline: 2.2628x; 2.1244x over previous
import jax
import jax.numpy as jnp
from jax.experimental import pallas as pl
from jax.experimental.pallas import tpu as pltpu

B = 64
H = 512
W = 512
SIZE = 511
N = SIZE * SIZE
OUT = 128
RT = 8
C = RT * SIZE
G = 64


def _fused_kernel(wc_ref, x_ref, wfc_ref, b_ref, out_ref, xcarry_ref, acc_ref):
    i = pl.program_id(0)
    j = (G - 1) - i

    @pl.when(i == 0)
    def _init():
        xcarry_ref[...] = jnp.zeros_like(xcarry_ref)
        acc_ref[...] = jnp.zeros_like(acc_ref)

    wcv = wc_ref[...]
    w00 = wcv[0, 0]
    w01 = wcv[0, 1]
    w10 = wcv[0, 2]
    w11 = wcv[0, 3]

    xcarry = xcarry_ref[...]

    def conv_row(top, bot):
        return jnp.tanh(w00 * top[:, :SIZE] + w01 * top[:, 1:]
                        + w10 * bot[:, :SIZE] + w11 * bot[:, 1:])

    rows = [x_ref[:, d, :] for d in range(RT)] + [xcarry]
    y = [conv_row(rows[d], rows[d + 1]) for d in range(RT)]
    chunk = jnp.concatenate(y, axis=1)

    wblk = wfc_ref[...]

    @pl.when(i == 0)
    def _acc_masked():
        row = jax.lax.broadcasted_iota(jnp.int32, (C, 1), 0)
        wm = jnp.where(row < (N - C * j), wblk, 0.0)
        acc_ref[...] += jax.lax.dot_general(
            chunk, wm, (((1,), (0,)), ((), ())),
            preferred_element_type=jnp.float32)

    @pl.when(i != 0)
    def _acc():
        acc_ref[...] += jax.lax.dot_general(
            chunk, wblk, (((1,), (0,)), ((), ())),
            preferred_element_type=jnp.float32)

    xcarry_ref[...] = x_ref[:, 0, :]

    @pl.when(i == G - 1)
    def _finalize():
        out_ref[...] = jnp.tanh(acc_ref[...] + b_ref[...])


def kernel(x, W_conv, W_fc, b_fc):
    wc = W_conv.reshape(1, 4)
    b2 = b_fc.reshape(1, OUT)
    wfc_t = W_fc.T
    return pl.pallas_call(
        _fused_kernel,
        grid=(G,),
        in_specs=[
            pl.BlockSpec((1, 4), lambda i: (0, 0)),
            pl.BlockSpec((B, RT, W), lambda i: (0, G - 1 - i, 0)),
            pl.BlockSpec((C, OUT), lambda i: (G - 1 - i, 0)),
            pl.BlockSpec((1, OUT), lambda i: (0, 0)),
        ],
        out_specs=pl.BlockSpec((B, OUT), lambda i: (0, 0)),
        out_shape=jax.ShapeDtypeStruct((B, OUT), jnp.float32),
        scratch_shapes=[
            pltpu.VMEM((B, W), jnp.float32),
            pltpu.VMEM((B, OUT), jnp.float32),
        ],
    )(wc, x, wfc_t, b2)


# speedup vs baseline: 2.9195x; 1.2902x over previous
import jax
import jax.numpy as jnp
from jax.experimental import pallas as pl
from jax.experimental.pallas import tpu as pltpu

B = 64
H = 512
W = 512
SIZE = 511
N = SIZE * SIZE
OUT = 128
RT = 8
C = RT * SIZE
G = 64


def _fused_kernel(wc_ref, x_ref, wfc_ref, b_ref, out_ref, xcarry_ref, acc_ref):
    i = pl.program_id(0)
    j = (G - 1) - i

    @pl.when(i == 0)
    def _init():
        xcarry_ref[...] = jnp.zeros_like(xcarry_ref)
        acc_ref[...] = jnp.zeros_like(acc_ref)

    wcv = wc_ref[...]
    w00 = wcv[0, 0]
    w01 = wcv[0, 1]
    w10 = wcv[0, 2]
    w11 = wcv[0, 3]

    xcarry = xcarry_ref[...]

    def conv_row(top, bot):
        a = w00 * top + w10 * bot
        b = w01 * top + w11 * bot
        return jnp.tanh(a[:, :SIZE] + b[:, 1:])

    rows = [x_ref[:, d, :] for d in range(RT)] + [xcarry]
    y = [conv_row(rows[d], rows[d + 1]) for d in range(RT)]
    chunk = jnp.concatenate(y, axis=1)

    wblk = wfc_ref[...]

    @pl.when(i == 0)
    def _acc_masked():
        row = jax.lax.broadcasted_iota(jnp.int32, (C, 1), 0)
        wm = jnp.where(row < (N - C * j), wblk, 0.0)
        acc_ref[...] += jax.lax.dot_general(
            chunk, wm, (((1,), (0,)), ((), ())),
            preferred_element_type=jnp.float32)

    @pl.when(i != 0)
    def _acc():
        acc_ref[...] += jax.lax.dot_general(
            chunk, wblk, (((1,), (0,)), ((), ())),
            preferred_element_type=jnp.float32)

    xcarry_ref[...] = x_ref[:, 0, :]

    @pl.when(i == G - 1)
    def _finalize():
        out_ref[...] = jnp.tanh(acc_ref[...] + b_ref[...])


def kernel(x, W_conv, W_fc, b_fc):
    wc = W_conv.reshape(1, 4)
    b2 = b_fc.reshape(1, OUT)
    wfc_t = W_fc.T
    return pl.pallas_call(
        _fused_kernel,
        grid=(G,),
        in_specs=[
            pl.BlockSpec((1, 4), lambda i: (0, 0)),
            pl.BlockSpec((B, RT, W), lambda i: (0, G - 1 - i, 0)),
            pl.BlockSpec((C, OUT), lambda i: (G - 1 - i, 0)),
            pl.BlockSpec((1, OUT), lambda i: (0, 0)),
        ],
        out_specs=pl.BlockSpec((B, OUT), lambda i: (0, 0)),
        out_shape=jax.ShapeDtypeStruct((B, OUT), jnp.float32),
        scratch_shapes=[
            pltpu.VMEM((B, W), jnp.float32),
            pltpu.VMEM((B, OUT), jnp.float32),
        ],
    )(wc, x, wfc_t, b2)


# speedup vs baseline: 3.4991x; 1.1985x over previous
import jax
import jax.numpy as jnp
from jax.experimental import pallas as pl
from jax.experimental.pallas import tpu as pltpu

B = 64
H = 512
W = 512
SIZE = 511
N = SIZE * SIZE
OUT = 128
RT = 16
C = RT * SIZE
G = H // RT


def _fused_kernel(wc_ref, x_ref, wfc_ref, b_ref, out_ref, xcarry_ref, acc_ref):
    i = pl.program_id(0)
    j = (G - 1) - i

    @pl.when(i == 0)
    def _init():
        xcarry_ref[...] = jnp.zeros_like(xcarry_ref)
        acc_ref[...] = jnp.zeros_like(acc_ref)

    wcv = wc_ref[...]
    w00 = wcv[0, 0]
    w01 = wcv[0, 1]
    w10 = wcv[0, 2]
    w11 = wcv[0, 3]

    xcarry = xcarry_ref[...]

    def conv_row(top, bot):
        a = w00 * top + w10 * bot
        b = w01 * top + w11 * bot
        return jnp.tanh(a[:, :SIZE] + b[:, 1:])

    rows = [x_ref[:, d, :] for d in range(RT)] + [xcarry]
    y = [conv_row(rows[d], rows[d + 1]) for d in range(RT)]
    chunk = jnp.concatenate(y, axis=1)

    wblk = wfc_ref[...]

    @pl.when(i == 0)
    def _acc_masked():
        row = jax.lax.broadcasted_iota(jnp.int32, (C, 1), 0)
        wm = jnp.where(row < (N - C * j), wblk, 0.0)
        acc_ref[...] += jax.lax.dot_general(
            chunk, wm, (((1,), (0,)), ((), ())),
            preferred_element_type=jnp.float32)

    @pl.when(i != 0)
    def _acc():
        acc_ref[...] += jax.lax.dot_general(
            chunk, wblk, (((1,), (0,)), ((), ())),
            preferred_element_type=jnp.float32)

    xcarry_ref[...] = x_ref[:, 0, :]

    @pl.when(i == G - 1)
    def _finalize():
        out_ref[...] = jnp.tanh(acc_ref[...] + b_ref[...])


def kernel(x, W_conv, W_fc, b_fc):
    wc = W_conv.reshape(1, 4)
    b2 = b_fc.reshape(1, OUT)
    wfc_t = W_fc.T
    return pl.pallas_call(
        _fused_kernel,
        grid=(G,),
        in_specs=[
            pl.BlockSpec((1, 4), lambda i: (0, 0)),
            pl.BlockSpec((B, RT, W), lambda i: (0, G - 1 - i, 0)),
            pl.BlockSpec((C, OUT), lambda i: (G - 1 - i, 0)),
            pl.BlockSpec((1, OUT), lambda i: (0, 0)),
        ],
        out_specs=pl.BlockSpec((B, OUT), lambda i: (0, 0)),
        out_shape=jax.ShapeDtypeStruct((B, OUT), jnp.float32),
        scratch_shapes=[
            pltpu.VMEM((B, W), jnp.float32),
            pltpu.VMEM((B, OUT), jnp.float32),
        ],
    )(wc, x, wfc_t, b2)


# speedup vs baseline: 3.7139x; 1.0614x over previous
import jax
import jax.numpy as jnp
from jax.experimental import pallas as pl
from jax.experimental.pallas import tpu as pltpu

B = 64
H = 512
W = 512
SIZE = 511
N = SIZE * SIZE
OUT = 128
RT = 32
C = RT * SIZE
G = H // RT


def _fused_kernel(wc_ref, x_ref, wfc_ref, b_ref, out_ref, xcarry_ref, acc_ref):
    i = pl.program_id(0)
    j = (G - 1) - i

    @pl.when(i == 0)
    def _init():
        xcarry_ref[...] = jnp.zeros_like(xcarry_ref)
        acc_ref[...] = jnp.zeros_like(acc_ref)

    wcv = wc_ref[...]
    w00 = wcv[0, 0]
    w01 = wcv[0, 1]
    w10 = wcv[0, 2]
    w11 = wcv[0, 3]

    xcarry = xcarry_ref[...]

    def conv_row(top, bot):
        a = w00 * top + w10 * bot
        b = w01 * top + w11 * bot
        return jnp.tanh(a[:, :SIZE] + b[:, 1:])

    rows = [x_ref[:, d, :] for d in range(RT)] + [xcarry]
    y = [conv_row(rows[d], rows[d + 1]) for d in range(RT)]
    chunk = jnp.concatenate(y, axis=1)

    wblk = wfc_ref[...]

    @pl.when(i == 0)
    def _acc_masked():
        row = jax.lax.broadcasted_iota(jnp.int32, (C, 1), 0)
        wm = jnp.where(row < (N - C * j), wblk, 0.0)
        acc_ref[...] += jax.lax.dot_general(
            chunk, wm, (((1,), (0,)), ((), ())),
            preferred_element_type=jnp.float32)

    @pl.when(i != 0)
    def _acc():
        acc_ref[...] += jax.lax.dot_general(
            chunk, wblk, (((1,), (0,)), ((), ())),
            preferred_element_type=jnp.float32)

    xcarry_ref[...] = x_ref[:, 0, :]

    @pl.when(i == G - 1)
    def _finalize():
        out_ref[...] = jnp.tanh(acc_ref[...] + b_ref[...])


def kernel(x, W_conv, W_fc, b_fc):
    wc = W_conv.reshape(1, 4)
    b2 = b_fc.reshape(1, OUT)
    wfc_t = W_fc.T
    return pl.pallas_call(
        _fused_kernel,
        grid=(G,),
        in_specs=[
            pl.BlockSpec((1, 4), lambda i: (0, 0)),
            pl.BlockSpec((B, RT, W), lambda i: (0, G - 1 - i, 0)),
            pl.BlockSpec((C, OUT), lambda i: (G - 1 - i, 0)),
            pl.BlockSpec((1, OUT), lambda i: (0, 0)),
        ],
        out_specs=pl.BlockSpec((B, OUT), lambda i: (0, 0)),
        out_shape=jax.ShapeDtypeStruct((B, OUT), jnp.float32),
        scratch_shapes=[
            pltpu.VMEM((B, W), jnp.float32),
            pltpu.VMEM((B, OUT), jnp.float32),
        ],
    )(wc, x, wfc_t, b2)
